# Initial kernel scaffold; baseline (speedup 1.0000x reference)
import jax
import jax.numpy as jnp
from jax.experimental import pallas as pl


def kernel(x, W):
    raise NotImplementedError("write your pallas kernel here")


# speedup vs baseline: 99.7375x; 99.7375x over previous
import functools

import jax
import jax.numpy as jnp
from jax.experimental import pallas as pl
from jax.experimental.pallas import tpu as pltpu

_SEQ = 512
_MAX_REL = 32
_TABLE = 2 * _MAX_REL + 1


def _rpe_kernel(x_ref, w_ref, o_ref, *, tile_s, seq_len):
    s0 = pl.program_id(0) * tile_s
    i = s0 + jax.lax.broadcasted_iota(jnp.int32, (tile_s, _TABLE), 0)
    k = jax.lax.broadcasted_iota(jnp.int32, (tile_s, _TABLE), 1)
    pos = i + k - _MAX_REL
    interior = ((pos >= 0) & (pos < seq_len)).astype(jnp.float32)
    low = jnp.maximum(i - (_MAX_REL - 1), 0).astype(jnp.float32)
    high = jnp.maximum((seq_len - _MAX_REL) - i, 0).astype(jnp.float32)
    c = jnp.where(k == 0, low, jnp.where(k == _TABLE - 1, high, interior))
    bias = jnp.dot(c, w_ref[...], preferred_element_type=jnp.float32)
    bias = bias * (1.0 / seq_len)
    o_ref[...] = x_ref[...] + bias[:, None, :]


def kernel(x, W):
    seq_len, batch, d_model = x.shape
    tile_s = 64
    grid = (seq_len // tile_s,)
    out = pl.pallas_call(
        functools.partial(_rpe_kernel, tile_s=tile_s, seq_len=seq_len),
        grid=grid,
        in_specs=[
            pl.BlockSpec((tile_s, batch, d_model), lambda s: (s, 0, 0)),
            pl.BlockSpec((_TABLE, d_model), lambda s: (0, 0)),
        ],
        out_specs=pl.BlockSpec((tile_s, batch, d_model), lambda s: (s, 0, 0)),
        out_shape=jax.ShapeDtypeStruct((seq_len, batch, d_model), x.dtype),
        compiler_params=pltpu.CompilerParams(
            dimension_semantics=("arbitrary",),
        ),
    )(x, W)
    return out


# speedup vs baseline: 108.1138x; 1.0840x over previous
import functools

import jax
import jax.numpy as jnp
from jax.experimental import pallas as pl
from jax.experimental.pallas import tpu as pltpu

_SEQ = 512
_MAX_REL = 32
_TABLE = 2 * _MAX_REL + 1


def _rpe_kernel(x_ref, w_ref, o_ref, *, tile_s, seq_len):
    s0 = pl.program_id(0) * tile_s
    i = s0 + jax.lax.broadcasted_iota(jnp.int32, (tile_s, _TABLE), 0)
    k = jax.lax.broadcasted_iota(jnp.int32, (tile_s, _TABLE), 1)
    pos = i + k - _MAX_REL
    interior = ((pos >= 0) & (pos < seq_len)).astype(jnp.float32)
    low = jnp.maximum(i - (_MAX_REL - 1), 0).astype(jnp.float32)
    high = jnp.maximum((seq_len - _MAX_REL) - i, 0).astype(jnp.float32)
    c = jnp.where(k == 0, low, jnp.where(k == _TABLE - 1, high, interior))
    bias = jnp.dot(c, w_ref[...], preferred_element_type=jnp.float32)
    bias = bias * (1.0 / seq_len)
    o_ref[...] = x_ref[...] + bias[:, None, :]


def kernel(x, W):
    seq_len, batch, d_model = x.shape
    tile_s = 128
    grid = (seq_len // tile_s,)
    out = pl.pallas_call(
        functools.partial(_rpe_kernel, tile_s=tile_s, seq_len=seq_len),
        grid=grid,
        in_specs=[
            pl.BlockSpec((tile_s, batch, d_model), lambda s: (s, 0, 0)),
            pl.BlockSpec((_TABLE, d_model), lambda s: (0, 0)),
        ],
        out_specs=pl.BlockSpec((tile_s, batch, d_model), lambda s: (s, 0, 0)),
        out_shape=jax.ShapeDtypeStruct((seq_len, batch, d_model), x.dtype),
        compiler_params=pltpu.CompilerParams(
            dimension_semantics=("arbitrary",),
        ),
    )(x, W)
    return out


# speedup vs baseline: 128.0755x; 1.1846x over previous
import functools

import jax
import jax.numpy as jnp
from jax.experimental import pallas as pl
from jax.experimental.pallas import tpu as pltpu

_SEQ = 512
_MAX_REL = 32
_TABLE = 2 * _MAX_REL + 1


def _rpe_kernel(x_ref, w_ref, o_ref, *, tile_s, seq_len):
    s0 = pl.program_id(0) * tile_s
    i = s0 + jax.lax.broadcasted_iota(jnp.int32, (tile_s, _TABLE), 0)
    k = jax.lax.broadcasted_iota(jnp.int32, (tile_s, _TABLE), 1)
    pos = i + k - _MAX_REL
    interior = ((pos >= 0) & (pos < seq_len)).astype(jnp.float32)
    low = jnp.maximum(i - (_MAX_REL - 1), 0).astype(jnp.float32)
    high = jnp.maximum((seq_len - _MAX_REL) - i, 0).astype(jnp.float32)
    c = jnp.where(k == 0, low, jnp.where(k == _TABLE - 1, high, interior))
    bias = jnp.dot(c, w_ref[...], preferred_element_type=jnp.float32)
    bias = bias * (1.0 / seq_len)
    o_ref[...] = x_ref[...] + bias[:, None, :]


def kernel(x, W):
    seq_len, batch, d_model = x.shape
    tile_s = 256
    grid = (seq_len // tile_s,)
    out = pl.pallas_call(
        functools.partial(_rpe_kernel, tile_s=tile_s, seq_len=seq_len),
        grid=grid,
        in_specs=[
            pl.BlockSpec((tile_s, batch, d_model), lambda s: (s, 0, 0)),
            pl.BlockSpec((_TABLE, d_model), lambda s: (0, 0)),
        ],
        out_specs=pl.BlockSpec((tile_s, batch, d_model), lambda s: (s, 0, 0)),
        out_shape=jax.ShapeDtypeStruct((seq_len, batch, d_model), x.dtype),
        compiler_params=pltpu.CompilerParams(
            dimension_semantics=("arbitrary",),
        ),
    )(x, W)
    return out
